# Initial kernel scaffold; baseline (speedup 1.0000x reference)
import jax
import jax.numpy as jnp
from jax.experimental import pallas as pl


def kernel(input_tensor, values):
    raise NotImplementedError("write your pallas kernel here")


# speedup vs baseline: 1.3305x; 1.3305x over previous
import numpy as np
import jax
import jax.numpy as jnp
from jax import lax
from jax.experimental import pallas as pl

_HASH_SEED = 1
_DEP = 5
_SLOTS = 5120
_PRIME = 2147483647
_BATCH = 4096
_ROWS = _DEP * _BATCH
_BB = 256


def _hash_tables():
    rng = np.random.RandomState(_HASH_SEED)
    A = rng.randint(1, _PRIME, size=(_DEP,)).astype(np.int64)
    B = rng.randint(0, _PRIME, size=(_DEP,)).astype(np.int64)
    v = np.arange(1024, dtype=np.int64)
    T0 = (A[:, None] * v[None, :] + B[:, None]) % _PRIME
    T1 = (A[:, None] * 1024 * v[None, :]) % _PRIME
    return T0.astype(np.uint32), T1.astype(np.uint32)


_T0, _T1 = _hash_tables()


def _onehot_rows_kernel(slot_ref, val_ref, out_ref):
    s = slot_ref[0, 0, :]
    v = val_ref[0, 0, :]
    iota = lax.broadcasted_iota(jnp.int32, (_BB, _SLOTS), 1)
    out_ref[...] = jnp.where(iota == s[:, None], v[:, None], 0.0)


def kernel(input_tensor, values):
    x = input_tensor.astype(jnp.int32)
    x1 = (x >> 10).astype(jnp.int32)
    x0 = (x & 1023).astype(jnp.int32)
    t0 = jnp.asarray(_T0)[:, :]
    t1 = jnp.asarray(_T1)
    s = t1[jnp.arange(_DEP)[:, None], x1[None, :]] + t0[jnp.arange(_DEP)[:, None], x0[None, :]]
    r = jnp.where(s >= jnp.uint32(_PRIME), s - jnp.uint32(_PRIME), s)
    slot_k = (r.astype(jnp.int32)) % _SLOTS
    slot_row = slot_k.reshape(-1).reshape(_BATCH, _DEP).T.reshape(-1)
    val_row = values.astype(jnp.float32).reshape(_BATCH, _DEP).T.reshape(-1)

    nblk = _ROWS // _BB
    out = pl.pallas_call(
        _onehot_rows_kernel,
        grid=(nblk,),
        in_specs=[
            pl.BlockSpec((1, 1, _BB), lambda i: (i, i * 0, i * 0)),
            pl.BlockSpec((1, 1, _BB), lambda i: (i, i * 0, i * 0)),
        ],
        out_specs=pl.BlockSpec((_BB, _SLOTS), lambda i: (i, i * 0)),
        out_shape=jax.ShapeDtypeStruct((_ROWS, _SLOTS), jnp.float32),
    )(slot_row.reshape(nblk, 1, _BB), val_row.reshape(nblk, 1, _BB))
    return out.reshape(_DEP, _BATCH, _SLOTS)


# speedup vs baseline: 1.3307x; 1.0002x over previous
import numpy as np
import jax
import jax.numpy as jnp
from jax import lax
from jax.experimental import pallas as pl
from jax.experimental.pallas import tpu as pltpu

_HASH_SEED = 1
_DEP = 5
_SLOTS = 5120
_PRIME = 2147483647
_BATCH = 4096
_ROWS = _DEP * _BATCH

_CB = 64
_NBUF = 8
_NCHUNK = _ROWS // _CB


def _hash_tables():
    rng = np.random.RandomState(_HASH_SEED)
    A = rng.randint(1, _PRIME, size=(_DEP,)).astype(np.int64)
    B = rng.randint(0, _PRIME, size=(_DEP,)).astype(np.int64)
    v = np.arange(1024, dtype=np.int64)
    T0 = (A[:, None] * v[None, :] + B[:, None]) % _PRIME
    T1 = (A[:, None] * 1024 * v[None, :]) % _PRIME
    return T0.astype(np.uint32), T1.astype(np.uint32)


_T0, _T1 = _hash_tables()


def _onehot_stream_kernel(slot_ref, val_ref, out_ref, ring_ref, sems):
    def chunk(i, _):
        j = lax.rem(i, jnp.int32(_NBUF))

        @pl.when(i >= jnp.int32(_NBUF))
        def _wait_prior():
            pltpu.make_async_copy(
                ring_ref.at[j],
                out_ref.at[pl.ds((i - jnp.int32(_NBUF)) * jnp.int32(_CB), _CB)],
                sems.at[j],
            ).wait()

        s = slot_ref[i, 0, :]
        v = val_ref[i, 0, :]
        iota = lax.broadcasted_iota(jnp.int32, (_CB, _SLOTS), 1)
        ring_ref[j] = jnp.where(iota == s[:, None], v[:, None], 0.0)

        pltpu.make_async_copy(
            ring_ref.at[j],
            out_ref.at[pl.ds(i * jnp.int32(_CB), _CB)],
            sems.at[j],
        ).start()
        return jnp.int32(0)

    lax.fori_loop(jnp.int32(0), jnp.int32(_NCHUNK), chunk, jnp.int32(0))

    def drain(i, _):
        j = lax.rem(i, jnp.int32(_NBUF))
        pltpu.make_async_copy(
            ring_ref.at[j],
            out_ref.at[pl.ds(i * jnp.int32(_CB), _CB)],
            sems.at[j],
        ).wait()
        return jnp.int32(0)

    lax.fori_loop(jnp.int32(_NCHUNK - _NBUF), jnp.int32(_NCHUNK), drain,
                  jnp.int32(0))


def kernel(input_tensor, values):
    x = input_tensor.astype(jnp.int32)
    x1 = (x >> 10).astype(jnp.int32)
    x0 = (x & 1023).astype(jnp.int32)
    t0 = jnp.asarray(_T0)
    t1 = jnp.asarray(_T1)
    dep = jnp.arange(_DEP)[:, None]
    s = t1[dep, x1[None, :]] + t0[dep, x0[None, :]]
    r = jnp.where(s >= jnp.uint32(_PRIME), s - jnp.uint32(_PRIME), s)
    slot_k = (r.astype(jnp.int32)) % _SLOTS
    slot_row = slot_k.reshape(-1).reshape(_BATCH, _DEP).T.reshape(-1)
    val_row = values.astype(jnp.float32).reshape(_BATCH, _DEP).T.reshape(-1)

    out = pl.pallas_call(
        _onehot_stream_kernel,
        in_specs=[
            pl.BlockSpec(memory_space=pltpu.MemorySpace.VMEM),
            pl.BlockSpec(memory_space=pltpu.MemorySpace.VMEM),
        ],
        out_specs=pl.BlockSpec(memory_space=pltpu.MemorySpace.HBM),
        out_shape=jax.ShapeDtypeStruct((_ROWS, _SLOTS), jnp.float32),
        scratch_shapes=[
            pltpu.VMEM((_NBUF, _CB, _SLOTS), jnp.float32),
            pltpu.SemaphoreType.DMA((_NBUF,)),
        ],
    )(slot_row.reshape(_NCHUNK, 1, _CB), val_row.reshape(_NCHUNK, 1, _CB))
    return out.reshape(_DEP, _BATCH, _SLOTS)
